# baseline (speedup 1.0000x reference)
import math

import jax
import jax.numpy as jnp
from jax.experimental import pallas as pl

_BLK = 1024
_FINE = 64


def _sinusoid_block_kernel(fine_ref, out_ref):
    h = fine_ref.shape[1] // 2
    sub = _BLK // _FINE
    p0 = pl.program_id(0) * _BLK

    col = jax.lax.broadcasted_iota(jnp.int32, (sub, h), 1).astype(jnp.float32)
    row = jax.lax.broadcasted_iota(jnp.int32, (sub, h), 0).astype(jnp.float32)
    freq = jnp.exp((col * (1.0 / h)) * (-math.log(10000.0)))
    ang = (jnp.float32(p0) + row * jnp.float32(_FINE)) * freq
    cs = jnp.sin(ang)[:, None, :]
    cc = jnp.cos(ang)[:, None, :]

    fs = fine_ref[:, :h][None, :, :]
    fc = fine_ref[:, h:][None, :, :]

    out_ref[0, :, :] = jnp.concatenate(
        [
            (fs * cc + fc * cs).reshape(_BLK, h),
            (fc * cc - fs * cs).reshape(_BLK, h),
        ],
        axis=1,
    )


def kernel(x, emb):
    seq_len = x.shape[1]
    hidden = emb.shape[1]
    grid = seq_len // _BLK
    return pl.pallas_call(
        _sinusoid_block_kernel,
        grid=(grid,),
        in_specs=[
            pl.BlockSpec((_FINE, hidden), lambda i: (0, 0)),
        ],
        out_specs=pl.BlockSpec((1, _BLK, hidden), lambda i: (0, i, 0)),
        out_shape=jax.ShapeDtypeStruct((1, seq_len, hidden), emb.dtype),
    )(emb)


# speedup vs baseline: 1.0581x; 1.0581x over previous
import math

import jax
import jax.numpy as jnp
from jax.experimental import pallas as pl

_BLK = 512
_FINE = 64


def _sinusoid_block_kernel(fine_ref, out_ref):
    h = fine_ref.shape[1] // 2
    sub = _BLK // _FINE
    p0 = pl.program_id(0) * _BLK

    col = jax.lax.broadcasted_iota(jnp.int32, (sub, h), 1).astype(jnp.float32)
    row = jax.lax.broadcasted_iota(jnp.int32, (sub, h), 0).astype(jnp.float32)
    freq = jnp.exp((col * (1.0 / h)) * (-math.log(10000.0)))
    ang = (jnp.float32(p0) + row * jnp.float32(_FINE)) * freq
    cs = jnp.sin(ang)[:, None, :]
    cc = jnp.cos(ang)[:, None, :]

    fs = fine_ref[:, :h][None, :, :]
    fc = fine_ref[:, h:][None, :, :]

    out_ref[0, :, :] = jnp.concatenate(
        [
            (fs * cc + fc * cs).reshape(_BLK, h),
            (fc * cc - fs * cs).reshape(_BLK, h),
        ],
        axis=1,
    )


def kernel(x, emb):
    seq_len = x.shape[1]
    hidden = emb.shape[1]
    grid = seq_len // _BLK
    return pl.pallas_call(
        _sinusoid_block_kernel,
        grid=(grid,),
        in_specs=[
            pl.BlockSpec((_FINE, hidden), lambda i: (0, 0)),
        ],
        out_specs=pl.BlockSpec((1, _BLK, hidden), lambda i: (0, i, 0)),
        out_shape=jax.ShapeDtypeStruct((1, seq_len, hidden), emb.dtype),
    )(emb)
